# Initial kernel scaffold; baseline (speedup 1.0000x reference)
import jax
import jax.numpy as jnp
from jax.experimental import pallas as pl


def kernel(x, ln_w, ln_b, w, b):
    raise NotImplementedError("write your pallas kernel here")


# speedup vs baseline: 2.2231x; 2.2231x over previous
import functools

import jax
import jax.numpy as jnp
from jax.experimental import pallas as pl

_EPS = 1e-5
_DROP_RATIO = 0.25


def _score_kernel(x_ref, lnw_ref, lnb_ref, w_ref, b_ref, out_ref):
    xb = x_ref[0]
    c = xb.shape[1]
    lnw = lnw_ref[...]
    lnb = lnb_ref[...]
    wr = w_ref[...]
    wp = lnw * wr
    ones = jnp.ones_like(wp)
    w2 = jnp.concatenate([wp, ones], axis=0)
    y = jax.lax.dot_general(xb, w2, (((1,), (1,)), ((), ())),
                            preferred_element_type=jnp.float32)
    s2 = jax.lax.dot_general(xb * xb, ones, (((1,), (1,)), ((), ())),
                             preferred_element_type=jnp.float32)
    dot = y[:, 0:1]
    mean = y[:, 1:2] * (1.0 / c)
    var = s2 * (1.0 / c) - mean * mean
    inv = jax.lax.rsqrt(var + _EPS)
    sum_wp = jnp.sum(wp, axis=1, keepdims=True)
    off = jnp.sum(lnb * wr, axis=1, keepdims=True) + b_ref[...]
    score = (dot - mean * sum_wp) * inv + off
    out_ref[...] = score[:, 0][None, None, :]


def _mask_kernel(s_ref, o_ref, *, k):
    s = s_ref[...]
    n = s.shape[1]
    i32 = jax.lax.bitcast_convert_type(s, jnp.int32)
    key = jnp.where(i32 >= 0, i32, i32 ^ jnp.int32(0x7FFFFFFF))

    lo = jnp.full((s.shape[0], 1), jnp.iinfo(jnp.int32).min, jnp.int32)
    hi = jnp.full((s.shape[0], 1), jnp.iinfo(jnp.int32).max, jnp.int32)
    for _ in range(32):
        mid = (lo & hi) + ((lo ^ hi) >> 1)
        cnt = jnp.sum((key <= mid).astype(jnp.int32), axis=1, keepdims=True)
        pred = cnt >= k
        hi = jnp.where(pred, mid, hi)
        lo = jnp.where(pred, lo, mid + 1)
    t = lo

    less = key < t
    eq = key == t
    c_less = jnp.sum(less.astype(jnp.int32), axis=1, keepdims=True)
    r = k - c_less
    idx = jax.lax.broadcasted_iota(jnp.int32, s.shape, 1)
    lo2 = jnp.zeros_like(c_less)
    hi2 = jnp.full_like(c_less, n)
    for _ in range(13):
        mid = (lo2 + hi2) >> 1
        cnt = jnp.sum((eq & (idx < mid)).astype(jnp.int32), axis=1,
                      keepdims=True)
        pred = cnt >= r
        hi2 = jnp.where(pred, mid, hi2)
        lo2 = jnp.where(pred, lo2, mid + 1)
    m = lo2

    selected = less | (eq & (idx < m))
    o_ref[...] = jnp.where(selected, -jnp.inf, 0.0).astype(jnp.float32)


def kernel(x, ln_w, ln_b, w, b):
    B, N, C = x.shape
    k = int(round(N * _DROP_RATIO))
    BN = 512
    lnw2 = ln_w.reshape(1, C)
    lnb2 = ln_b.reshape(1, C)
    w2 = w.reshape(1, C)
    b2 = b.reshape(1, 1)

    NB = N // BN
    scores = pl.pallas_call(
        _score_kernel,
        grid=(B, NB),
        in_specs=[
            pl.BlockSpec((1, BN, C), lambda i, j: (i, j, 0)),
            pl.BlockSpec((1, C), lambda i, j: (0, 0)),
            pl.BlockSpec((1, C), lambda i, j: (0, 0)),
            pl.BlockSpec((1, C), lambda i, j: (0, 0)),
            pl.BlockSpec((1, 1), lambda i, j: (0, 0)),
        ],
        out_specs=pl.BlockSpec((1, 1, BN), lambda i, j: (i * (N // BN) + j, 0, 0)),
        out_shape=jax.ShapeDtypeStruct((B * NB, 1, BN), jnp.float32),
    )(x, lnw2, lnb2, w2, b2)
    scores = scores.reshape(B, N)

    mask = pl.pallas_call(
        functools.partial(_mask_kernel, k=k),
        out_shape=jax.ShapeDtypeStruct((B, N), jnp.float32),
    )(scores)
    return mask[..., None]


# speedup vs baseline: 2.4345x; 1.0951x over previous
import functools

import jax
import jax.numpy as jnp
from jax.experimental import pallas as pl

_EPS = 1e-5
_DROP_RATIO = 0.25


def _score_kernel(x_ref, lnw_ref, lnb_ref, w_ref, b_ref, out_ref):
    xb = x_ref[0]
    c = xb.shape[1]
    lnw = lnw_ref[...]
    lnb = lnb_ref[...]
    wr = w_ref[...]
    wp = lnw * wr
    ones = jnp.ones_like(wp)
    w2 = jnp.concatenate([wp, ones], axis=0)
    y = jax.lax.dot_general(xb, w2, (((1,), (1,)), ((), ())),
                            preferred_element_type=jnp.float32)
    s2 = jax.lax.dot_general(xb * xb, ones, (((1,), (1,)), ((), ())),
                             preferred_element_type=jnp.float32)
    dot = y[:, 0:1]
    mean = y[:, 1:2] * (1.0 / c)
    var = s2 * (1.0 / c) - mean * mean
    inv = jax.lax.rsqrt(var + _EPS)
    sum_wp = jnp.sum(wp, axis=1, keepdims=True)
    off = jnp.sum(lnb * wr, axis=1, keepdims=True) + b_ref[...]
    score = (dot - mean * sum_wp) * inv + off
    out_ref[...] = score[:, 0][None, None, :]


def _mask_kernel(s_ref, o_ref, *, k):
    s = s_ref[...]
    n = s.shape[1]
    i32 = jax.lax.bitcast_convert_type(s, jnp.int32)
    key = jnp.where(i32 >= 0, i32, i32 ^ jnp.int32(0x7FFFFFFF))

    lo = jnp.full((s.shape[0], 1), jnp.iinfo(jnp.int32).min, jnp.int32)
    hi = jnp.full((s.shape[0], 1), jnp.iinfo(jnp.int32).max, jnp.int32)
    for _ in range(32):
        mid = (lo & hi) + ((lo ^ hi) >> 1)
        cnt = jnp.sum((key <= mid).astype(jnp.int32), axis=1, keepdims=True)
        pred = cnt >= k
        hi = jnp.where(pred, mid, hi)
        lo = jnp.where(pred, lo, mid + 1)
    t = lo

    less = key < t
    eq = key == t
    c_less = jnp.sum(less.astype(jnp.int32), axis=1, keepdims=True)
    r = k - c_less
    idx = jax.lax.broadcasted_iota(jnp.int32, s.shape, 1)
    lo2 = jnp.zeros_like(c_less)
    hi2 = jnp.full_like(c_less, n)
    for _ in range(13):
        mid = (lo2 + hi2) >> 1
        cnt = jnp.sum((eq & (idx < mid)).astype(jnp.int32), axis=1,
                      keepdims=True)
        pred = cnt >= r
        hi2 = jnp.where(pred, mid, hi2)
        lo2 = jnp.where(pred, lo2, mid + 1)
    m = lo2

    selected = less | (eq & (idx < m))
    o_ref[...] = jnp.where(selected, -jnp.inf, 0.0).astype(jnp.float32)


def kernel(x, ln_w, ln_b, w, b):
    B, N, C = x.shape
    k = int(round(N * _DROP_RATIO))
    BN = 1024
    lnw2 = ln_w.reshape(1, C)
    lnb2 = ln_b.reshape(1, C)
    w2 = w.reshape(1, C)
    b2 = b.reshape(1, 1)

    NB = N // BN
    scores = pl.pallas_call(
        _score_kernel,
        grid=(B, NB),
        in_specs=[
            pl.BlockSpec((1, BN, C), lambda i, j: (i, j, 0)),
            pl.BlockSpec((1, C), lambda i, j: (0, 0)),
            pl.BlockSpec((1, C), lambda i, j: (0, 0)),
            pl.BlockSpec((1, C), lambda i, j: (0, 0)),
            pl.BlockSpec((1, 1), lambda i, j: (0, 0)),
        ],
        out_specs=pl.BlockSpec((1, 1, BN), lambda i, j: (i * (N // BN) + j, 0, 0)),
        out_shape=jax.ShapeDtypeStruct((B * NB, 1, BN), jnp.float32),
    )(x, lnw2, lnb2, w2, b2)
    scores = scores.reshape(B, N)

    mask = pl.pallas_call(
        functools.partial(_mask_kernel, k=k),
        out_shape=jax.ShapeDtypeStruct((B, N), jnp.float32),
    )(scores)
    return mask[..., None]


# speedup vs baseline: 2.4732x; 1.0159x over previous
import functools

import jax
import jax.numpy as jnp
from jax.experimental import pallas as pl

_EPS = 1e-5
_DROP_RATIO = 0.25


def _score_kernel(x_ref, lnw_ref, lnb_ref, w_ref, b_ref, out_ref):
    xb = x_ref[0]
    c = xb.shape[1]
    lnw = lnw_ref[...]
    lnb = lnb_ref[...]
    wr = w_ref[...]
    wp = lnw * wr
    ones = jnp.ones_like(wp)
    w2 = jnp.concatenate([wp, ones], axis=0)
    y = jax.lax.dot_general(xb, w2, (((1,), (1,)), ((), ())),
                            preferred_element_type=jnp.float32)
    s2 = jax.lax.dot_general(xb * xb, ones, (((1,), (1,)), ((), ())),
                             preferred_element_type=jnp.float32)
    dot = y[:, 0:1]
    mean = y[:, 1:2] * (1.0 / c)
    var = s2 * (1.0 / c) - mean * mean
    inv = jax.lax.rsqrt(var + _EPS)
    sum_wp = jnp.sum(wp, axis=1, keepdims=True)
    off = jnp.sum(lnb * wr, axis=1, keepdims=True) + b_ref[...]
    score = (dot - mean * sum_wp) * inv + off
    out_ref[...] = score[:, 0][None, None, :]


def _mask_kernel(s_ref, o_ref, *, k):
    s = s_ref[...]
    n = s.shape[1]
    i32 = jax.lax.bitcast_convert_type(s, jnp.int32)
    key = jnp.where(i32 >= 0, i32, i32 ^ jnp.int32(0x7FFFFFFF))

    lo = jnp.full((s.shape[0], 1), jnp.iinfo(jnp.int32).min, jnp.int32)
    hi = jnp.full((s.shape[0], 1), jnp.iinfo(jnp.int32).max, jnp.int32)
    for _ in range(32):
        mid = (lo & hi) + ((lo ^ hi) >> 1)
        cnt = jnp.sum((key <= mid).astype(jnp.int32), axis=1, keepdims=True)
        pred = cnt >= k
        hi = jnp.where(pred, mid, hi)
        lo = jnp.where(pred, lo, mid + 1)
    t = lo

    less = key < t
    eq = key == t
    c_less = jnp.sum(less.astype(jnp.int32), axis=1, keepdims=True)
    r = k - c_less
    idx = jax.lax.broadcasted_iota(jnp.int32, s.shape, 1)
    lo2 = jnp.zeros_like(c_less)
    hi2 = jnp.full_like(c_less, n)
    for _ in range(13):
        mid = (lo2 + hi2) >> 1
        cnt = jnp.sum((eq & (idx < mid)).astype(jnp.int32), axis=1,
                      keepdims=True)
        pred = cnt >= r
        hi2 = jnp.where(pred, mid, hi2)
        lo2 = jnp.where(pred, lo2, mid + 1)
    m = lo2

    selected = less | (eq & (idx < m))
    o_ref[...] = jnp.where(selected, -jnp.inf, 0.0).astype(jnp.float32)


def kernel(x, ln_w, ln_b, w, b):
    B, N, C = x.shape
    k = int(round(N * _DROP_RATIO))
    BN = 2048
    lnw2 = ln_w.reshape(1, C)
    lnb2 = ln_b.reshape(1, C)
    w2 = w.reshape(1, C)
    b2 = b.reshape(1, 1)

    NB = N // BN
    scores = pl.pallas_call(
        _score_kernel,
        grid=(B, NB),
        in_specs=[
            pl.BlockSpec((1, BN, C), lambda i, j: (i, j, 0)),
            pl.BlockSpec((1, C), lambda i, j: (0, 0)),
            pl.BlockSpec((1, C), lambda i, j: (0, 0)),
            pl.BlockSpec((1, C), lambda i, j: (0, 0)),
            pl.BlockSpec((1, 1), lambda i, j: (0, 0)),
        ],
        out_specs=pl.BlockSpec((1, 1, BN), lambda i, j: (i * (N // BN) + j, 0, 0)),
        out_shape=jax.ShapeDtypeStruct((B * NB, 1, BN), jnp.float32),
    )(x, lnw2, lnb2, w2, b2)
    scores = scores.reshape(B, N)

    mask = pl.pallas_call(
        functools.partial(_mask_kernel, k=k),
        out_shape=jax.ShapeDtypeStruct((B, N), jnp.float32),
    )(scores)
    return mask[..., None]


# speedup vs baseline: 2.5345x; 1.0248x over previous
import functools

import jax
import jax.numpy as jnp
from jax.experimental import pallas as pl

_EPS = 1e-5
_DROP_RATIO = 0.25


def _score_kernel(x_ref, lnw_ref, lnb_ref, w_ref, b_ref, out_ref):
    xb = x_ref[0]
    c = xb.shape[1]
    lnw = lnw_ref[...]
    lnb = lnb_ref[...]
    wr = w_ref[...]
    wp = lnw * wr
    ones = jnp.ones_like(wp)
    w2 = jnp.concatenate([wp, ones], axis=0)
    y = jax.lax.dot_general(xb, w2, (((1,), (1,)), ((), ())),
                            preferred_element_type=jnp.float32)
    s2 = jax.lax.dot_general(xb * xb, ones, (((1,), (1,)), ((), ())),
                             preferred_element_type=jnp.float32)
    dot = y[:, 0:1]
    mean = y[:, 1:2] * (1.0 / c)
    var = s2 * (1.0 / c) - mean * mean
    inv = jax.lax.rsqrt(var + _EPS)
    sum_wp = jnp.sum(wp, axis=1, keepdims=True)
    off = jnp.sum(lnb * wr, axis=1, keepdims=True) + b_ref[...]
    score = (dot - mean * sum_wp) * inv + off
    out_ref[...] = score[:, 0][None, None, :]


def _mask_kernel(s_ref, o_ref, *, k):
    s = s_ref[...]
    n = s.shape[1]
    i32 = jax.lax.bitcast_convert_type(s, jnp.int32)
    key = jnp.where(i32 >= 0, i32, i32 ^ jnp.int32(0x7FFFFFFF))

    def avg(a, b):
        return (a & b) + ((a ^ b) >> 1)

    def cnt_le(m):
        return jnp.sum((key <= m).astype(jnp.int32), axis=1, keepdims=True)

    lo = jnp.full((s.shape[0], 1), jnp.iinfo(jnp.int32).min, jnp.int32)
    hi = jnp.full((s.shape[0], 1), jnp.iinfo(jnp.int32).max, jnp.int32)
    for _ in range(16):
        m2 = avg(lo, hi)
        m1 = avg(lo, m2)
        m3 = avg(m2 + 1, hi)
        p1 = cnt_le(m1) >= k
        p2 = cnt_le(m2) >= k
        p3 = cnt_le(m3) >= k
        hi = jnp.where(p1, m1, jnp.where(p2, m2, jnp.where(p3, m3, hi)))
        lo = jnp.where(p1, lo,
                       jnp.where(p2, m1 + 1, jnp.where(p3, m2 + 1, m3 + 1)))
    t = lo

    less = key < t
    eq = key == t
    c_less = jnp.sum(less.astype(jnp.int32), axis=1, keepdims=True)
    r = k - c_less
    idx = jax.lax.broadcasted_iota(jnp.int32, s.shape, 1)

    def cnt_eq_below(m):
        return jnp.sum((eq & (idx < m)).astype(jnp.int32), axis=1,
                       keepdims=True)

    lo2 = jnp.zeros_like(c_less)
    hi2 = jnp.full_like(c_less, n)
    for _ in range(7):
        m2 = (lo2 + hi2) >> 1
        m1 = (lo2 + m2) >> 1
        m3 = (m2 + 1 + hi2) >> 1
        p1 = cnt_eq_below(m1) >= r
        p2 = cnt_eq_below(m2) >= r
        p3 = cnt_eq_below(m3) >= r
        hi2 = jnp.where(p1, m1, jnp.where(p2, m2, jnp.where(p3, m3, hi2)))
        lo2 = jnp.where(p1, lo2,
                        jnp.where(p2, m1 + 1, jnp.where(p3, m2 + 1, m3 + 1)))
    m = lo2

    selected = less | (eq & (idx < m))
    o_ref[...] = jnp.where(selected, -jnp.inf, 0.0).astype(jnp.float32)


def kernel(x, ln_w, ln_b, w, b):
    B, N, C = x.shape
    k = int(round(N * _DROP_RATIO))
    BN = 2048
    lnw2 = ln_w.reshape(1, C)
    lnb2 = ln_b.reshape(1, C)
    w2 = w.reshape(1, C)
    b2 = b.reshape(1, 1)

    NB = N // BN
    scores = pl.pallas_call(
        _score_kernel,
        grid=(B, NB),
        in_specs=[
            pl.BlockSpec((1, BN, C), lambda i, j: (i, j, 0)),
            pl.BlockSpec((1, C), lambda i, j: (0, 0)),
            pl.BlockSpec((1, C), lambda i, j: (0, 0)),
            pl.BlockSpec((1, C), lambda i, j: (0, 0)),
            pl.BlockSpec((1, 1), lambda i, j: (0, 0)),
        ],
        out_specs=pl.BlockSpec((1, 1, BN), lambda i, j: (i * (N // BN) + j, 0, 0)),
        out_shape=jax.ShapeDtypeStruct((B * NB, 1, BN), jnp.float32),
    )(x, lnw2, lnb2, w2, b2)
    scores = scores.reshape(B, N)

    mask = pl.pallas_call(
        functools.partial(_mask_kernel, k=k),
        out_shape=jax.ShapeDtypeStruct((B, N), jnp.float32),
    )(scores)
    return mask[..., None]


# speedup vs baseline: 2.6752x; 1.0555x over previous
import functools

import jax
import jax.numpy as jnp
from jax.experimental import pallas as pl
from jax.experimental.pallas import tpu as pltpu

_EPS = 1e-5
_DROP_RATIO = 0.25


def _mask_from_scores(s, k):
    n = s.shape[1]
    i32 = jax.lax.bitcast_convert_type(s, jnp.int32)
    key = jnp.where(i32 >= 0, i32, i32 ^ jnp.int32(0x7FFFFFFF))

    def avg(a, b):
        return (a & b) + ((a ^ b) >> 1)

    def cnt_le(m):
        return jnp.sum((key <= m).astype(jnp.int32), axis=1, keepdims=True)

    lo = jnp.full((s.shape[0], 1), jnp.iinfo(jnp.int32).min, jnp.int32)
    hi = jnp.full((s.shape[0], 1), jnp.iinfo(jnp.int32).max, jnp.int32)
    for _ in range(16):
        m2 = avg(lo, hi)
        m1 = avg(lo, m2)
        m3 = avg(m2 + 1, hi)
        p1 = cnt_le(m1) >= k
        p2 = cnt_le(m2) >= k
        p3 = cnt_le(m3) >= k
        hi = jnp.where(p1, m1, jnp.where(p2, m2, jnp.where(p3, m3, hi)))
        lo = jnp.where(p1, lo,
                       jnp.where(p2, m1 + 1, jnp.where(p3, m2 + 1, m3 + 1)))
    t = lo

    less = key < t
    eq = key == t
    c_less = jnp.sum(less.astype(jnp.int32), axis=1, keepdims=True)
    r = k - c_less
    idx = jax.lax.broadcasted_iota(jnp.int32, s.shape, 1)

    def cnt_eq_below(m):
        return jnp.sum((eq & (idx < m)).astype(jnp.int32), axis=1,
                       keepdims=True)

    lo2 = jnp.zeros_like(c_less)
    hi2 = jnp.full_like(c_less, n)
    for _ in range(7):
        m2 = (lo2 + hi2) >> 1
        m1 = (lo2 + m2) >> 1
        m3 = (m2 + 1 + hi2) >> 1
        p1 = cnt_eq_below(m1) >= r
        p2 = cnt_eq_below(m2) >= r
        p3 = cnt_eq_below(m3) >= r
        hi2 = jnp.where(p1, m1, jnp.where(p2, m2, jnp.where(p3, m3, hi2)))
        lo2 = jnp.where(p1, lo2,
                        jnp.where(p2, m1 + 1, jnp.where(p3, m2 + 1, m3 + 1)))
    m = lo2

    selected = less | (eq & (idx < m))
    return jnp.where(selected, -jnp.inf, 0.0).astype(jnp.float32)


def _fused_kernel(x_ref, lnw_ref, lnb_ref, w_ref, b_ref, o_ref, s_ref,
                  *, k, nb, bn):
    t = pl.program_id(0)
    bi = t // nb
    j = t % nb

    xb = x_ref[0]
    c = xb.shape[1]
    lnw = lnw_ref[...]
    lnb = lnb_ref[...]
    wr = w_ref[...]
    wp = lnw * wr
    ones = jnp.ones_like(wp)
    w2 = jnp.concatenate([wp, ones], axis=0)
    y = jax.lax.dot_general(xb, w2, (((1,), (1,)), ((), ())),
                            preferred_element_type=jnp.float32)
    s2 = jax.lax.dot_general(xb * xb, ones, (((1,), (1,)), ((), ())),
                             preferred_element_type=jnp.float32)
    dot = y[:, 0:1]
    mean = y[:, 1:2] * (1.0 / c)
    var = s2 * (1.0 / c) - mean * mean
    inv = jax.lax.rsqrt(var + _EPS)
    sum_wp = jnp.sum(wp, axis=1, keepdims=True)
    off = jnp.sum(lnb * wr, axis=1, keepdims=True) + b_ref[...]
    score = (dot - mean * sum_wp) * inv + off
    s_ref[pl.ds(bi, 1), pl.ds(j * bn, bn)] = score[:, 0][None, :]

    @pl.when(t == pl.num_programs(0) - 1)
    def _():
        o_ref[...] = _mask_from_scores(s_ref[...], k)


def kernel(x, ln_w, ln_b, w, b):
    B, N, C = x.shape
    k = int(round(N * _DROP_RATIO))
    BN = 2048
    NB = N // BN
    lnw2 = ln_w.reshape(1, C)
    lnb2 = ln_b.reshape(1, C)
    w2 = w.reshape(1, C)
    b2 = b.reshape(1, 1)

    mask = pl.pallas_call(
        functools.partial(_fused_kernel, k=k, nb=NB, bn=BN),
        grid=(B * NB,),
        in_specs=[
            pl.BlockSpec((1, BN, C), lambda t, nb=NB: (t // nb, t % nb, 0)),
            pl.BlockSpec((1, C), lambda t: (0, 0)),
            pl.BlockSpec((1, C), lambda t: (0, 0)),
            pl.BlockSpec((1, C), lambda t: (0, 0)),
            pl.BlockSpec((1, 1), lambda t: (0, 0)),
        ],
        out_specs=pl.BlockSpec((B, N), lambda t: (0, 0)),
        out_shape=jax.ShapeDtypeStruct((B, N), jnp.float32),
        scratch_shapes=[pltpu.VMEM((B, N), jnp.float32)],
    )(x, lnw2, lnb2, w2, b2)
    return mask[..., None]
